# baseline (speedup 1.0000x reference)
import functools

import jax
import jax.numpy as jnp
from jax.experimental import pallas as pl
from jax.experimental.pallas import tpu as pltpu

D_MODEL = 768
D_FF = 2048
N_EXPERTS = 16
TOP_K_EXPERTS = 2
N_PRIMITIVES = 8
TOP_K_PRIMITIVES = 4
TEMPERATURE = 1.0
T_TOKENS = 128
DEPTH = 4
SPLIT = 4


def _topk_softmax_coeffs(lg):
    E, P = lg.shape
    pidx = jax.lax.broadcasted_iota(jnp.int32, (E, P), 1)
    rem = lg
    sel = jnp.zeros((E, P), jnp.bool_)
    for _ in range(TOP_K_PRIMITIVES):
        mv = jnp.max(rem, axis=-1, keepdims=True)
        mi = jnp.min(jnp.where(rem >= mv, pidx, P), axis=-1, keepdims=True)
        pick = pidx == mi
        sel = jnp.logical_or(sel, pick)
        rem = jnp.where(pick, -jnp.inf, rem)
    zm = jnp.max(jnp.where(sel, lg, -jnp.inf), axis=-1, keepdims=True)
    w = jnp.where(sel, jnp.exp(lg - zm), 0.0)
    return w / jnp.sum(w, axis=-1, keepdims=True)


def _moe_kernel(xf_ref, rwt_ref, l1_ref, l2_ref, b1b_ref, b2b_ref,
                b1w_hbm, b2w_hbm,
                y_ref, aux_ref,
                ring1, ring2, sem1, sem2):
    P = N_PRIMITIVES
    E = N_EXPERTS
    T = T_TOKENS

    def cp1(p, s):
        c = D_MODEL // SPLIT
        sl = pl.ds(s * c, c)
        return pltpu.make_async_copy(b1w_hbm.at[p, sl], ring1.at[p % DEPTH, sl],
                                     sem1.at[p, s])

    def cp2(p, s):
        c = D_FF // SPLIT
        sl = pl.ds(s * c, c)
        return pltpu.make_async_copy(b2w_hbm.at[p, sl], ring2.at[p % DEPTH, sl],
                                     sem2.at[p, s])

    def start1(p):
        for s in range(SPLIT):
            cp1(p, s).start()

    def wait1(p):
        for s in range(SPLIT):
            cp1(p, s).wait()

    def start2(p):
        for s in range(SPLIT):
            cp2(p, s).start()

    def wait2(p):
        for s in range(SPLIT):
            cp2(p, s).wait()

    for j in range(DEPTH):
        start1(j)
    start2(0)

    xf = xf_ref[...]
    xb = xf.astype(jnp.bfloat16)
    logits = jnp.dot(xb, rwt_ref[...].astype(jnp.bfloat16),
                     preferred_element_type=jnp.float32)
    m = jnp.max(logits, axis=-1, keepdims=True)
    ex = jnp.exp(logits - m)
    probs = ex / jnp.sum(ex, axis=-1, keepdims=True)
    eidx = jax.lax.broadcasted_iota(jnp.int32, (T, E), 1)
    v1 = jnp.max(probs, axis=-1, keepdims=True)
    i1 = jnp.min(jnp.where(probs >= v1, eidx, E), axis=-1, keepdims=True)
    m1 = eidx == i1
    p2 = jnp.where(m1, -jnp.inf, probs)
    v2 = jnp.max(p2, axis=-1, keepdims=True)
    i2 = jnp.min(jnp.where(p2 >= v2, eidx, E), axis=-1, keepdims=True)
    m2 = eidx == i2
    denom = v1 + v2 + 1e-8
    g1 = v1 / denom
    g2 = v2 / denom
    oh1 = m1.astype(jnp.float32)
    oh2 = m2.astype(jnp.float32)
    counts = jnp.sum(oh1 + oh2, axis=0, keepdims=True)
    f = counts / (jnp.sum(counts, axis=-1, keepdims=True) + 1e-8)
    pm = jnp.mean(probs, axis=0, keepdims=True)
    aux_ref[...] = jnp.sum(f * pm, axis=-1, keepdims=True) * E
    a1 = _topk_softmax_coeffs(l1_ref[...] / TEMPERATURE)
    a2 = _topk_softmax_coeffs(l2_ref[...] / TEMPERATURE)
    b1e = jnp.dot(a1, b1b_ref[...], preferred_element_type=jnp.float32)
    b2e = jnp.dot(a2, b2b_ref[...], preferred_element_type=jnp.float32)

    ak1 = jnp.dot(oh1, a1, preferred_element_type=jnp.float32)
    ak2 = jnp.dot(oh2, a1, preferred_element_type=jnp.float32)
    mix1 = jnp.dot(oh1, b1e, preferred_element_type=jnp.float32)
    mix2 = jnp.dot(oh2, b1e, preferred_element_type=jnp.float32)
    bk1 = g1 * jnp.dot(oh1, a2, preferred_element_type=jnp.float32)
    bk2 = g2 * jnp.dot(oh2, a2, preferred_element_type=jnp.float32)
    for p in range(P):
        wait1(p)
        acc = jnp.dot(xb, ring1[p % DEPTH].astype(jnp.bfloat16),
                      preferred_element_type=jnp.float32)
        mix1 = mix1 + ak1[:, p:p + 1] * acc
        mix2 = mix2 + ak2[:, p:p + 1] * acc
        if p + DEPTH < P:
            start1(p + DEPTH)
        elif p + DEPTH - P + 1 < DEPTH:
            start2(p + DEPTH - P + 1)

    h1 = jax.nn.gelu(mix1)
    h2 = jax.nn.gelu(mix2)
    g = oh1 * g1 + oh2 * g2
    y = jnp.dot(g, b2e, preferred_element_type=jnp.float32)

    for p in range(P):
        up = (bk1[:, p:p + 1] * h1 + bk2[:, p:p + 1] * h2).astype(jnp.bfloat16)
        wait2(p)
        y = y + jnp.dot(up, ring2[p % DEPTH].astype(jnp.bfloat16),
                        preferred_element_type=jnp.float32)
        if p + DEPTH < P:
            start2(p + DEPTH)
    y_ref[...] = y


@jax.jit
def kernel(x, router_w, fc1_logits, fc2_logits, bank_fc1_w, bank_fc1_b,
           bank_fc2_w, bank_fc2_b):
    Bq, Sq, D = x.shape
    xf = x.reshape(-1, D)
    T = xf.shape[0]
    P = N_PRIMITIVES
    E = N_EXPERTS
    F = D_FF

    vmem = lambda: pl.BlockSpec(memory_space=pltpu.MemorySpace.VMEM)
    any_ = lambda: pl.BlockSpec(memory_space=pl.ANY)
    y, aux = pl.pallas_call(
        _moe_kernel,
        in_specs=[vmem(), vmem(), vmem(), vmem(), vmem(), vmem(),
                  any_(), any_()],
        out_specs=[vmem(), vmem()],
        out_shape=[
            jax.ShapeDtypeStruct((T, D), jnp.float32),
            jax.ShapeDtypeStruct((1, 1), jnp.float32),
        ],
        scratch_shapes=[
            pltpu.VMEM((DEPTH, D, F), jnp.float32),
            pltpu.VMEM((DEPTH, F, D), jnp.float32),
            pltpu.SemaphoreType.DMA((P, SPLIT)),
            pltpu.SemaphoreType.DMA((P, SPLIT)),
        ],
    )(xf, router_w.T, fc1_logits, fc2_logits, bank_fc1_b, bank_fc2_b,
      bank_fc1_w, bank_fc2_w)
    return y.reshape(Bq, Sq, D), aux[0, 0]


# speedup vs baseline: 1.0415x; 1.0415x over previous
import functools

import jax
import jax.numpy as jnp
from jax.experimental import pallas as pl
from jax.experimental.pallas import tpu as pltpu

D_MODEL = 768
D_FF = 2048
N_EXPERTS = 16
TOP_K_EXPERTS = 2
N_PRIMITIVES = 8
TOP_K_PRIMITIVES = 4
TEMPERATURE = 1.0
T_TOKENS = 128
DEPTH = 3
SPLIT = 1


def _topk_softmax_coeffs(lg):
    E, P = lg.shape
    pidx = jax.lax.broadcasted_iota(jnp.int32, (E, P), 1)
    rem = lg
    sel = jnp.zeros((E, P), jnp.bool_)
    for _ in range(TOP_K_PRIMITIVES):
        mv = jnp.max(rem, axis=-1, keepdims=True)
        mi = jnp.min(jnp.where(rem >= mv, pidx, P), axis=-1, keepdims=True)
        pick = pidx == mi
        sel = jnp.logical_or(sel, pick)
        rem = jnp.where(pick, -jnp.inf, rem)
    zm = jnp.max(jnp.where(sel, lg, -jnp.inf), axis=-1, keepdims=True)
    w = jnp.where(sel, jnp.exp(lg - zm), 0.0)
    return w / jnp.sum(w, axis=-1, keepdims=True)


def _moe_kernel(xf_ref, rwt_ref, l1_ref, l2_ref, b1b_ref, b2b_ref,
                b1w_hbm, b2w_hbm,
                y_ref, aux_ref,
                ring1, ring2, sem1, sem2):
    P = N_PRIMITIVES
    E = N_EXPERTS
    T = T_TOKENS

    def cp1(p, s):
        c = D_MODEL // SPLIT
        sl = pl.ds(s * c, c)
        return pltpu.make_async_copy(b1w_hbm.at[p, sl], ring1.at[p % DEPTH, sl],
                                     sem1.at[p, s])

    def cp2(p, s):
        c = D_FF // SPLIT
        sl = pl.ds(s * c, c)
        return pltpu.make_async_copy(b2w_hbm.at[p, sl], ring2.at[p % DEPTH, sl],
                                     sem2.at[p, s])

    def start1(p):
        for s in range(SPLIT):
            cp1(p, s).start()

    def wait1(p):
        for s in range(SPLIT):
            cp1(p, s).wait()

    def start2(p):
        for s in range(SPLIT):
            cp2(p, s).start()

    def wait2(p):
        for s in range(SPLIT):
            cp2(p, s).wait()

    for j in range(DEPTH):
        start1(j)
    start2(0)

    xf = xf_ref[...]
    xb = xf.astype(jnp.bfloat16)
    logits = jnp.dot(xb, rwt_ref[...].astype(jnp.bfloat16),
                     preferred_element_type=jnp.float32)
    m = jnp.max(logits, axis=-1, keepdims=True)
    ex = jnp.exp(logits - m)
    probs = ex / jnp.sum(ex, axis=-1, keepdims=True)
    eidx = jax.lax.broadcasted_iota(jnp.int32, (T, E), 1)
    v1 = jnp.max(probs, axis=-1, keepdims=True)
    i1 = jnp.min(jnp.where(probs >= v1, eidx, E), axis=-1, keepdims=True)
    m1 = eidx == i1
    p2 = jnp.where(m1, -jnp.inf, probs)
    v2 = jnp.max(p2, axis=-1, keepdims=True)
    i2 = jnp.min(jnp.where(p2 >= v2, eidx, E), axis=-1, keepdims=True)
    m2 = eidx == i2
    denom = v1 + v2 + 1e-8
    g1 = v1 / denom
    g2 = v2 / denom
    oh1 = m1.astype(jnp.float32)
    oh2 = m2.astype(jnp.float32)
    counts = jnp.sum(oh1 + oh2, axis=0, keepdims=True)
    f = counts / (jnp.sum(counts, axis=-1, keepdims=True) + 1e-8)
    pm = jnp.mean(probs, axis=0, keepdims=True)
    aux_ref[...] = jnp.sum(f * pm, axis=-1, keepdims=True) * E
    a1 = _topk_softmax_coeffs(l1_ref[...] / TEMPERATURE)
    a2 = _topk_softmax_coeffs(l2_ref[...] / TEMPERATURE)
    b1e = jnp.dot(a1, b1b_ref[...], preferred_element_type=jnp.float32)
    b2e = jnp.dot(a2, b2b_ref[...], preferred_element_type=jnp.float32)

    ak1 = jnp.dot(oh1, a1, preferred_element_type=jnp.float32)
    ak2 = jnp.dot(oh2, a1, preferred_element_type=jnp.float32)
    mix1 = jnp.dot(oh1, b1e, preferred_element_type=jnp.float32)
    mix2 = jnp.dot(oh2, b1e, preferred_element_type=jnp.float32)
    bk1 = g1 * jnp.dot(oh1, a2, preferred_element_type=jnp.float32)
    bk2 = g2 * jnp.dot(oh2, a2, preferred_element_type=jnp.float32)
    for p in range(P):
        wait1(p)
        acc = jnp.dot(xb, ring1[p % DEPTH].astype(jnp.bfloat16),
                      preferred_element_type=jnp.float32)
        mix1 = mix1 + ak1[:, p:p + 1] * acc
        mix2 = mix2 + ak2[:, p:p + 1] * acc
        if p + DEPTH < P:
            start1(p + DEPTH)
        elif p + DEPTH - P + 1 < DEPTH:
            start2(p + DEPTH - P + 1)

    h1 = jax.nn.gelu(mix1)
    h2 = jax.nn.gelu(mix2)
    g = oh1 * g1 + oh2 * g2
    y = jnp.dot(g, b2e, preferred_element_type=jnp.float32)

    for p in range(P):
        up = (bk1[:, p:p + 1] * h1 + bk2[:, p:p + 1] * h2).astype(jnp.bfloat16)
        wait2(p)
        y = y + jnp.dot(up, ring2[p % DEPTH].astype(jnp.bfloat16),
                        preferred_element_type=jnp.float32)
        if p + DEPTH < P:
            start2(p + DEPTH)
    y_ref[...] = y


@jax.jit
def kernel(x, router_w, fc1_logits, fc2_logits, bank_fc1_w, bank_fc1_b,
           bank_fc2_w, bank_fc2_b):
    Bq, Sq, D = x.shape
    xf = x.reshape(-1, D)
    T = xf.shape[0]
    P = N_PRIMITIVES
    E = N_EXPERTS
    F = D_FF

    vmem = lambda: pl.BlockSpec(memory_space=pltpu.MemorySpace.VMEM)
    any_ = lambda: pl.BlockSpec(memory_space=pl.ANY)
    y, aux = pl.pallas_call(
        _moe_kernel,
        in_specs=[vmem(), vmem(), vmem(), vmem(), vmem(), vmem(),
                  any_(), any_()],
        out_specs=[vmem(), vmem()],
        out_shape=[
            jax.ShapeDtypeStruct((T, D), jnp.float32),
            jax.ShapeDtypeStruct((1, 1), jnp.float32),
        ],
        scratch_shapes=[
            pltpu.VMEM((DEPTH, D, F), jnp.float32),
            pltpu.VMEM((DEPTH, F, D), jnp.float32),
            pltpu.SemaphoreType.DMA((P, SPLIT)),
            pltpu.SemaphoreType.DMA((P, SPLIT)),
        ],
    )(xf, router_w.T, fc1_logits, fc2_logits, bank_fc1_b, bank_fc2_b,
      bank_fc1_w, bank_fc2_w)
    return y.reshape(Bq, Sq, D), aux[0, 0]


# speedup vs baseline: 1.0630x; 1.0207x over previous
import functools

import jax
import jax.numpy as jnp
from jax.experimental import pallas as pl
from jax.experimental.pallas import tpu as pltpu

D_MODEL = 768
D_FF = 2048
N_EXPERTS = 16
TOP_K_EXPERTS = 2
N_PRIMITIVES = 8
TOP_K_PRIMITIVES = 4
TEMPERATURE = 1.0
T_TOKENS = 128
DEPTH = 3
SPLIT = 1


def _topk_softmax_coeffs(lg):
    E, P = lg.shape
    pidx = jax.lax.broadcasted_iota(jnp.int32, (E, P), 1)
    rem = lg
    sel = jnp.zeros((E, P), jnp.bool_)
    for _ in range(TOP_K_PRIMITIVES):
        mv = jnp.max(rem, axis=-1, keepdims=True)
        mi = jnp.min(jnp.where(rem >= mv, pidx, P), axis=-1, keepdims=True)
        pick = pidx == mi
        sel = jnp.logical_or(sel, pick)
        rem = jnp.where(pick, -jnp.inf, rem)
    zm = jnp.max(jnp.where(sel, lg, -jnp.inf), axis=-1, keepdims=True)
    w = jnp.where(sel, jnp.exp(lg - zm), 0.0)
    return w / jnp.sum(w, axis=-1, keepdims=True)


def _moe_kernel(xf_ref, rwt_ref, l1_ref, l2_ref, b1b_ref, b2b_ref,
                b1w_hbm, b2w_hbm,
                y_ref, aux_ref,
                ring1, ring2, sem1, sem2):
    P = N_PRIMITIVES
    E = N_EXPERTS
    T = T_TOKENS

    def cp1(p, s):
        c = D_MODEL // SPLIT
        sl = pl.ds(s * c, c)
        return pltpu.make_async_copy(b1w_hbm.at[p, sl], ring1.at[p % DEPTH, sl],
                                     sem1.at[p, s])

    def cp2(p, s):
        c = D_FF // SPLIT
        sl = pl.ds(s * c, c)
        return pltpu.make_async_copy(b2w_hbm.at[p, sl], ring2.at[p % DEPTH, sl],
                                     sem2.at[p, s])

    def start1(p):
        for s in range(SPLIT):
            cp1(p, s).start()

    def wait1(p):
        for s in range(SPLIT):
            cp1(p, s).wait()

    def start2(p):
        for s in range(SPLIT):
            cp2(p, s).start()

    def wait2(p):
        for s in range(SPLIT):
            cp2(p, s).wait()

    for j in range(DEPTH):
        start1(j)
    start2(0)

    xf = xf_ref[...]
    xb = xf.astype(jnp.bfloat16)
    logits = jax.lax.dot_general(
        xb, rwt_ref[...].astype(jnp.bfloat16),
        (((1,), (1,)), ((), ())),
        preferred_element_type=jnp.float32)
    m = jnp.max(logits, axis=-1, keepdims=True)
    ex = jnp.exp(logits - m)
    probs = ex / jnp.sum(ex, axis=-1, keepdims=True)
    eidx = jax.lax.broadcasted_iota(jnp.int32, (T, E), 1)
    v1 = jnp.max(probs, axis=-1, keepdims=True)
    i1 = jnp.min(jnp.where(probs >= v1, eidx, E), axis=-1, keepdims=True)
    m1 = eidx == i1
    p2 = jnp.where(m1, -jnp.inf, probs)
    v2 = jnp.max(p2, axis=-1, keepdims=True)
    i2 = jnp.min(jnp.where(p2 >= v2, eidx, E), axis=-1, keepdims=True)
    m2 = eidx == i2
    denom = v1 + v2 + 1e-8
    g1 = v1 / denom
    g2 = v2 / denom
    oh1 = m1.astype(jnp.float32)
    oh2 = m2.astype(jnp.float32)
    counts = jnp.sum(oh1 + oh2, axis=0, keepdims=True)
    f = counts / (jnp.sum(counts, axis=-1, keepdims=True) + 1e-8)
    pm = jnp.mean(probs, axis=0, keepdims=True)
    aux_ref[...] = jnp.sum(f * pm, axis=-1, keepdims=True) * E
    a1 = _topk_softmax_coeffs(l1_ref[...] / TEMPERATURE)
    a2 = _topk_softmax_coeffs(l2_ref[...] / TEMPERATURE)
    b1e = jnp.dot(a1, b1b_ref[...], preferred_element_type=jnp.float32)
    b2e = jnp.dot(a2, b2b_ref[...], preferred_element_type=jnp.float32)

    ak1 = jnp.dot(oh1, a1, preferred_element_type=jnp.float32)
    ak2 = jnp.dot(oh2, a1, preferred_element_type=jnp.float32)
    mix1 = jnp.dot(oh1, b1e, preferred_element_type=jnp.float32)
    mix2 = jnp.dot(oh2, b1e, preferred_element_type=jnp.float32)
    bk1 = g1 * jnp.dot(oh1, a2, preferred_element_type=jnp.float32)
    bk2 = g2 * jnp.dot(oh2, a2, preferred_element_type=jnp.float32)
    for p in range(P):
        wait1(p)
        acc = jnp.dot(xb, ring1[p % DEPTH].astype(jnp.bfloat16),
                      preferred_element_type=jnp.float32)
        mix1 = mix1 + ak1[:, p:p + 1] * acc
        mix2 = mix2 + ak2[:, p:p + 1] * acc
        if p + DEPTH < P:
            start1(p + DEPTH)
        elif p + DEPTH - P + 1 < DEPTH:
            start2(p + DEPTH - P + 1)

    h1 = jax.nn.gelu(mix1)
    h2 = jax.nn.gelu(mix2)
    g = oh1 * g1 + oh2 * g2
    y = jnp.dot(g, b2e, preferred_element_type=jnp.float32)

    for p in range(P):
        up = (bk1[:, p:p + 1] * h1 + bk2[:, p:p + 1] * h2).astype(jnp.bfloat16)
        wait2(p)
        y = y + jnp.dot(up, ring2[p % DEPTH].astype(jnp.bfloat16),
                        preferred_element_type=jnp.float32)
        if p + DEPTH < P:
            start2(p + DEPTH)
    y_ref[...] = y


@jax.jit
def kernel(x, router_w, fc1_logits, fc2_logits, bank_fc1_w, bank_fc1_b,
           bank_fc2_w, bank_fc2_b):
    Bq, Sq, D = x.shape
    xf = x.reshape(-1, D)
    T = xf.shape[0]
    P = N_PRIMITIVES
    E = N_EXPERTS
    F = D_FF

    vmem = lambda: pl.BlockSpec(memory_space=pltpu.MemorySpace.VMEM)
    any_ = lambda: pl.BlockSpec(memory_space=pl.ANY)
    y, aux = pl.pallas_call(
        _moe_kernel,
        in_specs=[vmem(), vmem(), vmem(), vmem(), vmem(), vmem(),
                  any_(), any_()],
        out_specs=[vmem(), vmem()],
        out_shape=[
            jax.ShapeDtypeStruct((T, D), jnp.float32),
            jax.ShapeDtypeStruct((1, 1), jnp.float32),
        ],
        scratch_shapes=[
            pltpu.VMEM((DEPTH, D, F), jnp.float32),
            pltpu.VMEM((DEPTH, F, D), jnp.float32),
            pltpu.SemaphoreType.DMA((P, SPLIT)),
            pltpu.SemaphoreType.DMA((P, SPLIT)),
        ],
    )(xf, router_w, fc1_logits, fc2_logits, bank_fc1_b, bank_fc2_b,
      bank_fc1_w, bank_fc2_w)
    return y.reshape(Bq, Sq, D), jnp.reshape(aux, ())


# speedup vs baseline: 1.0844x; 1.0202x over previous
import functools

import jax
import jax.numpy as jnp
from jax.experimental import pallas as pl
from jax.experimental.pallas import tpu as pltpu

D_MODEL = 768
D_FF = 2048
N_EXPERTS = 16
TOP_K_EXPERTS = 2
N_PRIMITIVES = 8
TOP_K_PRIMITIVES = 4
TEMPERATURE = 1.0
T_TOKENS = 128
DEPTH = 3
SPLIT = 1
DEPTH2 = 5


def _topk_softmax_coeffs(lg):
    E, P = lg.shape
    pidx = jax.lax.broadcasted_iota(jnp.int32, (E, P), 1)
    rem = lg
    sel = jnp.zeros((E, P), jnp.bool_)
    for _ in range(TOP_K_PRIMITIVES):
        mv = jnp.max(rem, axis=-1, keepdims=True)
        mi = jnp.min(jnp.where(rem >= mv, pidx, P), axis=-1, keepdims=True)
        pick = pidx == mi
        sel = jnp.logical_or(sel, pick)
        rem = jnp.where(pick, -jnp.inf, rem)
    zm = jnp.max(jnp.where(sel, lg, -jnp.inf), axis=-1, keepdims=True)
    w = jnp.where(sel, jnp.exp(lg - zm), 0.0)
    return w / jnp.sum(w, axis=-1, keepdims=True)


def _moe_kernel(xf_ref, rwt_ref, l1_ref, l2_ref, b1b_ref, b2b_ref,
                b1w_hbm, b2w_hbm,
                y_ref, aux_ref,
                ring1, ring2, sem1, sem2):
    P = N_PRIMITIVES
    E = N_EXPERTS
    T = T_TOKENS

    def cp1(p, s):
        c = D_MODEL // SPLIT
        sl = pl.ds(s * c, c)
        return pltpu.make_async_copy(b1w_hbm.at[p, sl], ring1.at[p % DEPTH, sl],
                                     sem1.at[p, s])

    HF = D_FF // 2

    def cp2(c, s):
        del s
        p, h = c // 2, c % 2
        return pltpu.make_async_copy(b2w_hbm.at[p, pl.ds(h * HF, HF)],
                                     ring2.at[c % DEPTH2], sem2.at[c, 0])

    def start1(p):
        for s in range(SPLIT):
            cp1(p, s).start()

    def wait1(p):
        for s in range(SPLIT):
            cp1(p, s).wait()

    def start2(c):
        cp2(c, 0).start()

    def wait2(c):
        cp2(c, 0).wait()

    for j in range(DEPTH):
        start1(j)
    start2(0)
    start2(1)

    xf = xf_ref[...]
    xb = xf.astype(jnp.bfloat16)
    logits = jax.lax.dot_general(
        xb, rwt_ref[...].astype(jnp.bfloat16),
        (((1,), (1,)), ((), ())),
        preferred_element_type=jnp.float32)
    m = jnp.max(logits, axis=-1, keepdims=True)
    ex = jnp.exp(logits - m)
    probs = ex / jnp.sum(ex, axis=-1, keepdims=True)
    eidx = jax.lax.broadcasted_iota(jnp.int32, (T, E), 1)
    v1 = jnp.max(probs, axis=-1, keepdims=True)
    i1 = jnp.min(jnp.where(probs >= v1, eidx, E), axis=-1, keepdims=True)
    m1 = eidx == i1
    p2 = jnp.where(m1, -jnp.inf, probs)
    v2 = jnp.max(p2, axis=-1, keepdims=True)
    i2 = jnp.min(jnp.where(p2 >= v2, eidx, E), axis=-1, keepdims=True)
    m2 = eidx == i2
    denom = v1 + v2 + 1e-8
    g1 = v1 / denom
    g2 = v2 / denom
    oh1 = m1.astype(jnp.float32)
    oh2 = m2.astype(jnp.float32)
    counts = jnp.sum(oh1 + oh2, axis=0, keepdims=True)
    f = counts / (jnp.sum(counts, axis=-1, keepdims=True) + 1e-8)
    pm = jnp.mean(probs, axis=0, keepdims=True)
    aux_ref[...] = jnp.sum(f * pm, axis=-1, keepdims=True) * E
    a1 = _topk_softmax_coeffs(l1_ref[...] / TEMPERATURE)
    a2 = _topk_softmax_coeffs(l2_ref[...] / TEMPERATURE)
    b1e = jnp.dot(a1, b1b_ref[...], preferred_element_type=jnp.float32)
    b2e = jnp.dot(a2, b2b_ref[...], preferred_element_type=jnp.float32)

    ak1 = jnp.dot(oh1, a1, preferred_element_type=jnp.float32)
    ak2 = jnp.dot(oh2, a1, preferred_element_type=jnp.float32)
    mix1 = jnp.dot(oh1, b1e, preferred_element_type=jnp.float32)
    mix2 = jnp.dot(oh2, b1e, preferred_element_type=jnp.float32)
    bk1 = g1 * jnp.dot(oh1, a2, preferred_element_type=jnp.float32)
    bk2 = g2 * jnp.dot(oh2, a2, preferred_element_type=jnp.float32)
    for p in range(P):
        wait1(p)
        acc = jnp.dot(xb, ring1[p % DEPTH].astype(jnp.bfloat16),
                      preferred_element_type=jnp.float32)
        mix1 = mix1 + ak1[:, p:p + 1] * acc
        mix2 = mix2 + ak2[:, p:p + 1] * acc
        if p + DEPTH < P:
            start1(p + DEPTH)
        else:
            start2(p + DEPTH - P + 2)

    h1 = jax.nn.gelu(mix1)
    h2 = jax.nn.gelu(mix2)
    g = oh1 * g1 + oh2 * g2
    y = jnp.dot(g, b2e, preferred_element_type=jnp.float32)

    for p in range(P):
        up = (bk1[:, p:p + 1] * h1 + bk2[:, p:p + 1] * h2).astype(jnp.bfloat16)
        for h in range(2):
            c = 2 * p + h
            wait2(c)
            y = y + jnp.dot(up[:, h * HF:(h + 1) * HF],
                            ring2[c % DEPTH2].astype(jnp.bfloat16),
                            preferred_element_type=jnp.float32)
            if c + DEPTH2 < 2 * P:
                start2(c + DEPTH2)
    y_ref[...] = y


@jax.jit
def kernel(x, router_w, fc1_logits, fc2_logits, bank_fc1_w, bank_fc1_b,
           bank_fc2_w, bank_fc2_b):
    Bq, Sq, D = x.shape
    xf = x.reshape(-1, D)
    T = xf.shape[0]
    P = N_PRIMITIVES
    E = N_EXPERTS
    F = D_FF

    vmem = lambda: pl.BlockSpec(memory_space=pltpu.MemorySpace.VMEM)
    any_ = lambda: pl.BlockSpec(memory_space=pl.ANY)
    y, aux = pl.pallas_call(
        _moe_kernel,
        in_specs=[vmem(), vmem(), vmem(), vmem(), vmem(), vmem(),
                  any_(), any_()],
        out_specs=[vmem(), vmem()],
        out_shape=[
            jax.ShapeDtypeStruct((T, D), jnp.float32),
            jax.ShapeDtypeStruct((1, 1), jnp.float32),
        ],
        scratch_shapes=[
            pltpu.VMEM((DEPTH, D, F), jnp.float32),
            pltpu.VMEM((DEPTH2, F // 2, D), jnp.float32),
            pltpu.SemaphoreType.DMA((P, SPLIT)),
            pltpu.SemaphoreType.DMA((2 * P, 1)),
        ],
    )(xf, router_w, fc1_logits, fc2_logits, bank_fc1_b, bank_fc2_b,
      bank_fc1_w, bank_fc2_w)
    return y.reshape(Bq, Sq, D), jnp.reshape(aux, ())


# speedup vs baseline: 1.0953x; 1.0101x over previous
import functools

import jax
import jax.numpy as jnp
from jax.experimental import pallas as pl
from jax.experimental.pallas import tpu as pltpu

D_MODEL = 768
D_FF = 2048
N_EXPERTS = 16
TOP_K_EXPERTS = 2
N_PRIMITIVES = 8
TOP_K_PRIMITIVES = 4
TEMPERATURE = 1.0
T_TOKENS = 128
DEPTH = 6
SPLIT = 1
DEPTH2 = 5


def _topk_softmax_coeffs(lg):
    E, P = lg.shape
    pidx = jax.lax.broadcasted_iota(jnp.int32, (E, P), 1)
    rem = lg
    sel = jnp.zeros((E, P), jnp.bool_)
    for _ in range(TOP_K_PRIMITIVES):
        mv = jnp.max(rem, axis=-1, keepdims=True)
        mi = jnp.min(jnp.where(rem >= mv, pidx, P), axis=-1, keepdims=True)
        pick = pidx == mi
        sel = jnp.logical_or(sel, pick)
        rem = jnp.where(pick, -jnp.inf, rem)
    zm = jnp.max(jnp.where(sel, lg, -jnp.inf), axis=-1, keepdims=True)
    w = jnp.where(sel, jnp.exp(lg - zm), 0.0)
    return w / jnp.sum(w, axis=-1, keepdims=True)


def _moe_kernel(xf_ref, rwt_ref, l1_ref, l2_ref, b1b_ref, b2b_ref,
                b1w_hbm, b2w_hbm,
                y_ref, aux_ref,
                ring1, ring2, sem1, sem2):
    P = N_PRIMITIVES
    E = N_EXPERTS
    T = T_TOKENS

    HD = D_MODEL // 2

    def cp1(c, s):
        del s
        p, h = c // 2, c % 2
        return pltpu.make_async_copy(b1w_hbm.at[p, pl.ds(h * HD, HD)],
                                     ring1.at[c % DEPTH], sem1.at[c, 0])

    HF = D_FF // 2

    def cp2(c, s):
        del s
        p, h = c // 2, c % 2
        return pltpu.make_async_copy(b2w_hbm.at[p, pl.ds(h * HF, HF)],
                                     ring2.at[c % DEPTH2], sem2.at[c, 0])

    def start1(c):
        cp1(c, 0).start()

    def wait1(c):
        cp1(c, 0).wait()

    def start2(c):
        cp2(c, 0).start()

    def wait2(c):
        cp2(c, 0).wait()

    for j in range(DEPTH):
        start1(j)
    start2(0)
    start2(1)

    xf = xf_ref[...]
    xb = xf.astype(jnp.bfloat16)
    logits = jax.lax.dot_general(
        xb, rwt_ref[...].astype(jnp.bfloat16),
        (((1,), (1,)), ((), ())),
        preferred_element_type=jnp.float32)
    m = jnp.max(logits, axis=-1, keepdims=True)
    ex = jnp.exp(logits - m)
    probs = ex / jnp.sum(ex, axis=-1, keepdims=True)
    eidx = jax.lax.broadcasted_iota(jnp.int32, (T, E), 1)
    v1 = jnp.max(probs, axis=-1, keepdims=True)
    i1 = jnp.min(jnp.where(probs >= v1, eidx, E), axis=-1, keepdims=True)
    m1 = eidx == i1
    p2 = jnp.where(m1, -jnp.inf, probs)
    v2 = jnp.max(p2, axis=-1, keepdims=True)
    i2 = jnp.min(jnp.where(p2 >= v2, eidx, E), axis=-1, keepdims=True)
    m2 = eidx == i2
    denom = v1 + v2 + 1e-8
    g1 = v1 / denom
    g2 = v2 / denom
    oh1 = m1.astype(jnp.float32)
    oh2 = m2.astype(jnp.float32)
    counts = jnp.sum(oh1 + oh2, axis=0, keepdims=True)
    f = counts / (jnp.sum(counts, axis=-1, keepdims=True) + 1e-8)
    pm = jnp.mean(probs, axis=0, keepdims=True)
    aux_ref[...] = jnp.sum(f * pm, axis=-1, keepdims=True) * E
    a1 = _topk_softmax_coeffs(l1_ref[...] / TEMPERATURE)
    a2 = _topk_softmax_coeffs(l2_ref[...] / TEMPERATURE)
    b1e = jnp.dot(a1, b1b_ref[...], preferred_element_type=jnp.float32)
    b2e = jnp.dot(a2, b2b_ref[...], preferred_element_type=jnp.float32)

    ak1 = jnp.dot(oh1, a1, preferred_element_type=jnp.float32)
    ak2 = jnp.dot(oh2, a1, preferred_element_type=jnp.float32)
    mix1 = jnp.dot(oh1, b1e, preferred_element_type=jnp.float32)
    mix2 = jnp.dot(oh2, b1e, preferred_element_type=jnp.float32)
    bk1 = g1 * jnp.dot(oh1, a2, preferred_element_type=jnp.float32)
    bk2 = g2 * jnp.dot(oh2, a2, preferred_element_type=jnp.float32)
    for p in range(P):
        acc = None
        for h in range(2):
            c = 2 * p + h
            wait1(c)
            part = jnp.dot(xb[:, h * HD:(h + 1) * HD],
                           ring1[c % DEPTH].astype(jnp.bfloat16),
                           preferred_element_type=jnp.float32)
            acc = part if acc is None else acc + part
            if c + DEPTH < 2 * P:
                start1(c + DEPTH)
            elif c >= 2 * P - 3:
                start2(c - (2 * P - 3) + 2)
        mix1 = mix1 + ak1[:, p:p + 1] * acc
        mix2 = mix2 + ak2[:, p:p + 1] * acc

    h1 = jax.nn.gelu(mix1)
    h2 = jax.nn.gelu(mix2)
    g = oh1 * g1 + oh2 * g2
    y = jnp.dot(g, b2e, preferred_element_type=jnp.float32)

    for p in range(P):
        up = (bk1[:, p:p + 1] * h1 + bk2[:, p:p + 1] * h2).astype(jnp.bfloat16)
        for h in range(2):
            c = 2 * p + h
            wait2(c)
            y = y + jnp.dot(up[:, h * HF:(h + 1) * HF],
                            ring2[c % DEPTH2].astype(jnp.bfloat16),
                            preferred_element_type=jnp.float32)
            if c + DEPTH2 < 2 * P:
                start2(c + DEPTH2)
    y_ref[...] = y


@jax.jit
def kernel(x, router_w, fc1_logits, fc2_logits, bank_fc1_w, bank_fc1_b,
           bank_fc2_w, bank_fc2_b):
    Bq, Sq, D = x.shape
    xf = x.reshape(-1, D)
    T = xf.shape[0]
    P = N_PRIMITIVES
    E = N_EXPERTS
    F = D_FF

    vmem = lambda: pl.BlockSpec(memory_space=pltpu.MemorySpace.VMEM)
    any_ = lambda: pl.BlockSpec(memory_space=pl.ANY)
    y, aux = pl.pallas_call(
        _moe_kernel,
        in_specs=[vmem(), vmem(), vmem(), vmem(), vmem(), vmem(),
                  any_(), any_()],
        out_specs=[vmem(), vmem()],
        out_shape=[
            jax.ShapeDtypeStruct((T, D), jnp.float32),
            jax.ShapeDtypeStruct((1, 1), jnp.float32),
        ],
        scratch_shapes=[
            pltpu.VMEM((DEPTH, D // 2, F), jnp.float32),
            pltpu.VMEM((DEPTH2, F // 2, D), jnp.float32),
            pltpu.SemaphoreType.DMA((2 * P, 1)),
            pltpu.SemaphoreType.DMA((2 * P, 1)),
        ],
    )(xf, router_w, fc1_logits, fc2_logits, bank_fc1_b, bank_fc2_b,
      bank_fc1_w, bank_fc2_w)
    return y.reshape(Bq, Sq, D), jnp.reshape(aux, ())
